# baseline (speedup 1.0000x reference)
import functools

import jax
import jax.numpy as jnp
import numpy as np
from jax import lax
from jax.experimental import pallas as pl
from jax.experimental.pallas import tpu as pltpu
from jax.experimental.pallas import tpu_sc as plsc

H = W = 96
NPIX = H * W
DIM_LOI = 128
DIM_FC = 1024
N_PTS0 = 32
N_PTS1 = 8
TOPK = 300


def _fc1_body(f_ref, w_ref, b_ref, o_ref):
    a = f_ref[...]
    w = w_ref[...]
    o = jax.lax.dot_general(a, w, (((0,), (1,)), ((), ())),
                            preferred_element_type=jnp.float32)
    o_ref[...] = o + b_ref[...][None, :]


def _fc1(features_flat, fc1_w, fc1_b):
    BN = 1024
    grid = (NPIX // BN,)
    return pl.pallas_call(
        _fc1_body,
        grid=grid,
        in_specs=[
            pl.BlockSpec((256, BN), lambda i: (0, i)),
            pl.BlockSpec((DIM_LOI, 256), lambda i: (0, 0)),
            pl.BlockSpec((DIM_LOI,), lambda i: (0,)),
        ],
        out_specs=pl.BlockSpec((BN, DIM_LOI), lambda i: (i, 0)),
        out_shape=jax.ShapeDtypeStruct((NPIX, DIM_LOI), jnp.float32),
    )(features_flat, fc1_w, fc1_b)


def _mlp_body(x_ref, w1_ref, b1_ref, w2_ref, b2_ref, w3_ref, b3_ref,
              keep_ref, o_ref):
    x = x_ref[...].astype(jnp.bfloat16)
    h = jax.lax.dot_general(x, w1_ref[...], (((1,), (1,)), ((), ())),
                            preferred_element_type=jnp.float32)
    h = jnp.maximum(h + b1_ref[...][None, :], 0.0).astype(jnp.bfloat16)
    h = jax.lax.dot_general(h, w2_ref[...], (((1,), (1,)), ((), ())),
                            preferred_element_type=jnp.float32)
    h = jnp.maximum(h + b2_ref[...][None, :], 0.0).astype(jnp.bfloat16)
    logits = jax.lax.dot_general(h, w3_ref[...], (((1,), (1,)), ((), ())),
                                 preferred_element_type=jnp.float32)
    logits = logits[:, 0:1] + b3_ref[0]
    o_ref[...] = jax.nn.sigmoid(logits) * keep_ref[...]


def _mlp(feats, w1, b1, w2, b2, w3, b3, keep):
    BM = 1024
    grid = (NPIX // BM,)
    return pl.pallas_call(
        _mlp_body,
        grid=grid,
        in_specs=[
            pl.BlockSpec((BM, DIM_FC), lambda i: (i, 0)),
            pl.BlockSpec((DIM_FC, DIM_FC), lambda i: (0, 0)),
            pl.BlockSpec((DIM_FC,), lambda i: (0,)),
            pl.BlockSpec((DIM_FC, DIM_FC), lambda i: (0, 0)),
            pl.BlockSpec((DIM_FC,), lambda i: (0,)),
            pl.BlockSpec((8, DIM_FC), lambda i: (0, 0)),
            pl.BlockSpec((1,), lambda i: (0,)),
            pl.BlockSpec((BM, 1), lambda i: (i, 0)),
        ],
        out_specs=pl.BlockSpec((BM, 1), lambda i: (i, 0)),
        out_shape=jax.ShapeDtypeStruct((NPIX, 1), jnp.float32),
    )(feats, w1, b1, w2, b2, w3, b3, keep)


def _maps_body(o_ref, lines_ref, nms_ref, joff_ref):
    o = o_ref[...]
    md0 = jax.nn.sigmoid(o[0])
    md1 = jax.nn.sigmoid(o[1])
    md2 = jax.nn.sigmoid(o[2])
    dis = jax.nn.sigmoid(o[3])
    m = jnp.maximum(o[5], o[6])
    e5 = jnp.exp(o[5] - m)
    e6 = jnp.exp(o[6] - m)
    jloc = e6 / (e5 + e6)
    joff_ref[0] = jax.nn.sigmoid(o[7]) - 0.5
    joff_ref[1] = jax.nn.sigmoid(o[8]) - 0.5

    y0 = jax.lax.broadcasted_iota(jnp.int32, (H, W), 0).astype(jnp.float32)
    x0 = jax.lax.broadcasted_iota(jnp.int32, (H, W), 1).astype(jnp.float32)
    md_ = (md0 - 0.5) * np.pi * 2.0
    st_ = md1 * np.pi / 2.0
    ed_ = -md2 * np.pi / 2.0
    cs_md = jnp.cos(md_)
    ss_md = jnp.sin(md_)
    cs_st = jnp.maximum(jnp.cos(st_), 0.001)
    ss_st = jnp.maximum(jnp.sin(st_), 0.001)
    cs_ed = jnp.maximum(jnp.cos(ed_), 0.001)
    ss_ed = jnp.minimum(jnp.sin(ed_), -0.001)
    y_st = ss_st / cs_st
    y_ed = ss_ed / cs_ed
    d = dis * 5.0
    lines_ref[0] = jnp.clip((cs_md - ss_md * y_st) * d + x0, 0.0, W - 1.0)
    lines_ref[1] = jnp.clip((ss_md + cs_md * y_st) * d + y0, 0.0, H - 1.0)
    lines_ref[2] = jnp.clip((cs_md - ss_md * y_ed) * d + x0, 0.0, W - 1.0)
    lines_ref[3] = jnp.clip((ss_md + cs_md * y_ed) * d + y0, 0.0, H - 1.0)

    ninf = jnp.float32(-jnp.inf)
    pad_row = jnp.full((1, W), ninf)
    up = jnp.concatenate([jloc[1:], pad_row], axis=0)
    dn = jnp.concatenate([pad_row, jloc[:-1]], axis=0)
    rm = jnp.maximum(jloc, jnp.maximum(up, dn))
    pad_col = jnp.full((H, 1), ninf)
    lf = jnp.concatenate([rm[:, 1:], pad_col], axis=1)
    rt = jnp.concatenate([pad_col, rm[:, :-1]], axis=1)
    ap = jnp.maximum(rm, jnp.maximum(lf, rt))
    nms_ref[...] = jloc * (jloc == ap).astype(jnp.float32)


def _maps(out0):
    return pl.pallas_call(
        _maps_body,
        out_shape=(
            jax.ShapeDtypeStruct((4, H, W), jnp.float32),
            jax.ShapeDtypeStruct((H, W), jnp.float32),
            jax.ShapeDtypeStruct((2, H, W), jnp.float32),
        ),
    )(out0)


def _topk_body(nms_ref, joff_ref, juncs_ref):
    fy = jax.lax.broadcasted_iota(jnp.int32, (H, W), 0)
    fx = jax.lax.broadcasted_iota(jnp.int32, (H, W), 1)
    flat = fy * W + fx
    joff0 = joff_ref[0]
    joff1 = joff_ref[1]
    big = jnp.int32(1 << 30)

    def step(k, v):
        mval = jnp.max(v)
        p = jnp.min(jnp.where(v == mval, flat, big))
        onehot = flat == p
        jox = jnp.sum(jnp.where(onehot, joff0, 0.0))
        joy = jnp.sum(jnp.where(onehot, joff1, 0.0))
        pxf = (p % W).astype(jnp.float32)
        pyf = (p // W).astype(jnp.float32)
        juncs_ref[0, k] = pxf + jox + 0.5
        juncs_ref[1, k] = pyf + joy + 0.5
        return jnp.where(onehot, -jnp.inf, v)

    lax.fori_loop(0, TOPK, step, nms_ref[...])


def _topk(nms, joff):
    return pl.pallas_call(
        _topk_body,
        out_shape=jax.ShapeDtypeStruct((2, 304), jnp.float32),
        out_specs=pl.BlockSpec(memory_space=pltpu.SMEM),
    )(nms, joff)


def _match_body(lines_ref, juncs_ref, uv_ref, keep_ref):
    l0 = lines_ref[0]
    l1 = lines_ref[1]
    l2 = lines_ref[2]
    l3 = lines_ref[3]
    inf = jnp.float32(jnp.inf)
    shp = (H, W)

    def step(j, c):
        d1m, i1, c1x, c1y, d2m, i2, c2x, c2y = c
        jx = juncs_ref[0, j]
        jy = juncs_ref[1, j]
        d1 = (l0 - jx) ** 2 + (l1 - jy) ** 2
        p1 = d1 < d1m
        d2 = (l2 - jx) ** 2 + (l3 - jy) ** 2
        p2 = d2 < d2m
        return (jnp.where(p1, d1, d1m), jnp.where(p1, j, i1),
                jnp.where(p1, jx, c1x), jnp.where(p1, jy, c1y),
                jnp.where(p2, d2, d2m), jnp.where(p2, j, i2),
                jnp.where(p2, jx, c2x), jnp.where(p2, jy, c2y))

    init = (jnp.full(shp, inf), jnp.zeros(shp, jnp.int32),
            jnp.zeros(shp), jnp.zeros(shp),
            jnp.full(shp, inf), jnp.zeros(shp, jnp.int32),
            jnp.zeros(shp), jnp.zeros(shp))
    _, i1, c1x, c1y, _, i2, c2x, c2y = lax.fori_loop(0, TOPK, step, init)
    swap = i2 < i1
    keep_ref[...] = (i1 != i2).astype(jnp.float32)
    uv_ref[0] = jnp.where(swap, c2x, c1x)
    uv_ref[1] = jnp.where(swap, c2y, c1y)
    uv_ref[2] = jnp.where(swap, c1x, c2x)
    uv_ref[3] = jnp.where(swap, c1y, c2y)


def _match(lines, juncs):
    return pl.pallas_call(
        _match_body,
        in_specs=[
            pl.BlockSpec(memory_space=pltpu.VMEM),
            pl.BlockSpec(memory_space=pltpu.SMEM),
        ],
        out_shape=(
            jax.ShapeDtypeStruct((4, H, W), jnp.float32),
            jax.ShapeDtypeStruct((H, W), jnp.float32),
        ),
    )(lines, juncs)


_NW = 32
_LPW = NPIX // _NW


def _sc_sample_body(uv_hbm, tsp_hbm, table_hbm, out_hbm,
                    uv_v, tsp_v, idx_v, g_v, o_v, sem_g0, sem_g1, sem_o):
    nc = 2
    wid = lax.axis_index("s") * nc + lax.axis_index("c")
    base = wid * _LPW
    for c4 in range(4):
        pltpu.sync_copy(uv_hbm.at[pl.ds(c4 * NPIX + base, _LPW)],
                        uv_v.at[pl.ds(c4 * _LPW, _LPW)])
    pltpu.sync_copy(tsp_hbm, tsp_v)
    sems = (sem_g0, sem_g1)

    def calc_idx(l, b):
        ux = uv_v[pl.ds(0 * _LPW + l, 16)][0]
        uy = uv_v[pl.ds(1 * _LPW + l, 16)][0]
        vx = uv_v[pl.ds(2 * _LPW + l, 16)][0]
        vy = uv_v[pl.ds(3 * _LPW + l, 16)][0]
        wts = []
        for g in range(2):
            t = tsp_v[pl.ds(g * 16, 16)]
            omt = 1.0 - t
            px = ux * t + vx * omt - 0.5
            py = uy * t + vy * omt - 0.5
            px0i = jnp.clip(px.astype(jnp.int32), 0, W - 1)
            py0i = jnp.clip(py.astype(jnp.int32), 0, H - 1)
            px0 = px0i.astype(jnp.float32)
            py0 = py0i.astype(jnp.float32)
            px1 = jnp.minimum(px0 + 1.0, W - 1.0)
            py1 = jnp.minimum(py0 + 1.0, H - 1.0)
            px1i = px1.astype(jnp.int32)
            py1i = py1.astype(jnp.int32)
            wx0 = px1 - px
            wx1 = px - px0
            wy0 = py1 - py
            wy1 = py - py0
            r0 = py0i * W
            r1 = py1i * W
            idx_v[b, pl.ds(0 * 32 + g * 16, 16)] = r0 + px0i
            idx_v[b, pl.ds(1 * 32 + g * 16, 16)] = r1 + px0i
            idx_v[b, pl.ds(2 * 32 + g * 16, 16)] = r0 + px1i
            idx_v[b, pl.ds(3 * 32 + g * 16, 16)] = r1 + px1i
            wts.append((wy0 * wx0, wy1 * wx0, wy0 * wx1, wy1 * wx1))
        return tuple(wts[0]) + tuple(wts[1])

    def start_gather(b):
        pltpu.async_copy(table_hbm.at[idx_v.at[b]], g_v.at[b], sems[b])

    def wait_gather(b):
        pltpu.make_async_copy(table_hbm.at[idx_v.at[b]], g_v.at[b],
                              sems[b]).wait()

    def compute(wts, b):
        for q in range(N_PTS1):
            acc = [None] * 8
            for k4 in range(4):
                pt = q * 4 + k4
                g, ln = pt // 16, pt % 16
                w00 = wts[4 * g][ln]
                w10 = wts[4 * g + 1][ln]
                w01 = wts[4 * g + 2][ln]
                w11 = wts[4 * g + 3][ln]
                for v in range(8):
                    cs = pl.ds(v * 16, 16)
                    val = (g_v[b, 0 * 32 + pt, cs] * w00
                           + g_v[b, 1 * 32 + pt, cs] * w10
                           + g_v[b, 2 * 32 + pt, cs] * w01
                           + g_v[b, 3 * 32 + pt, cs] * w11)
                    acc[v] = val if acc[v] is None else jnp.maximum(acc[v], val)
            for v in range(8):
                o_v[pl.ds(b * DIM_FC + q * 128 + v * 16, 16)] = acc[v]

    def out_copy(i):
        return pltpu.make_async_copy(
            o_v, out_hbm.at[pl.ds((base + 2 * i) * DIM_FC, 2 * DIM_FC)],
            sem_o)

    npair = _LPW // 2
    wts0_init = calc_idx(0, 0)
    start_gather(0)

    def pair_body(i, wts0):
        @pl.when(i > 0)
        def _():
            out_copy(i - 1).wait()
        wts1 = calc_idx(2 * i + 1, 1)
        start_gather(1)
        wait_gather(0)
        compute(wts0, 0)
        wts0n = calc_idx(2 * i + 2, 0)

        @pl.when(i < npair - 1)
        def _():
            start_gather(0)
        wait_gather(1)
        compute(wts1, 1)
        out_copy(i).start()
        return wts0n

    lax.fori_loop(0, npair, pair_body, wts0_init)
    out_copy(npair - 1).wait()


def _sc_sample(uv, tspan, table):
    mesh = plsc.VectorSubcoreMesh(core_axis_name="c", subcore_axis_name="s")
    f = pl.kernel(
        _sc_sample_body,
        out_type=jax.ShapeDtypeStruct((NPIX * DIM_FC,), jnp.float32),
        mesh=mesh,
        scratch_types=[
            pltpu.VMEM((4 * _LPW + 32,), jnp.float32),
            pltpu.VMEM((N_PTS0,), jnp.float32),
            pltpu.VMEM((2, 4 * N_PTS0), jnp.int32),
            pltpu.VMEM((2, 4 * N_PTS0, DIM_LOI), jnp.float32),
            pltpu.VMEM((2 * DIM_FC,), jnp.float32),
            pltpu.SemaphoreType.DMA,
            pltpu.SemaphoreType.DMA,
            pltpu.SemaphoreType.DMA,
        ],
    )
    return f(uv, tspan, table)


def kernel(output, features, fc1_w, fc1_b, w1, b1, w2, b2, w3, b3):
    features_flat = features[0].reshape(256, NPIX)
    loiT = _fc1(features_flat, fc1_w, fc1_b)

    lines, nms, joff = _maps(output[0])

    juncs = _topk(nms, joff)

    uv, iskeep = _match(lines, juncs)
    uv_flat = uv.reshape(-1)

    tspan = jnp.linspace(0.0, 1.0, N_PTS0)
    feats = _sc_sample(uv_flat, tspan, loiT).reshape(NPIX, DIM_FC)
    iskeep = iskeep.reshape(-1)

    w1p = w1.reshape(DIM_FC, DIM_LOI, N_PTS1).transpose(0, 2, 1).reshape(
        DIM_FC, DIM_FC).astype(jnp.bfloat16)
    w3p = jnp.concatenate([w3, jnp.zeros((7, DIM_FC), jnp.float32)],
                          axis=0).astype(jnp.bfloat16)

    scores_out = _mlp(feats, w1p, b1, w2.astype(jnp.bfloat16), b2, w3p, b3,
                      iskeep.reshape(NPIX, 1))
    return scores_out.reshape(-1)


# speedup vs baseline: 1.2643x; 1.2643x over previous
import functools

import jax
import jax.numpy as jnp
import numpy as np
from jax import lax
from jax.experimental import pallas as pl
from jax.experimental.pallas import tpu as pltpu
from jax.experimental.pallas import tpu_sc as plsc

H = W = 96
NPIX = H * W
DIM_LOI = 128
DIM_FC = 1024
N_PTS0 = 32
N_PTS1 = 8
TOPK = 300


def _fc1_body(f_ref, w_ref, b_ref, o_ref):
    a = f_ref[...]
    w = w_ref[...]
    o = jax.lax.dot_general(a, w, (((0,), (1,)), ((), ())),
                            preferred_element_type=jnp.float32)
    o_ref[...] = o + b_ref[...][None, :]


def _fc1(features_flat, fc1_w, fc1_b):
    BN = 1024
    grid = (NPIX // BN,)
    return pl.pallas_call(
        _fc1_body,
        grid=grid,
        in_specs=[
            pl.BlockSpec((256, BN), lambda i: (0, i)),
            pl.BlockSpec((DIM_LOI, 256), lambda i: (0, 0)),
            pl.BlockSpec((DIM_LOI,), lambda i: (0,)),
        ],
        out_specs=pl.BlockSpec((BN, DIM_LOI), lambda i: (i, 0)),
        out_shape=jax.ShapeDtypeStruct((NPIX, DIM_LOI), jnp.float32),
    )(features_flat, fc1_w, fc1_b)


def _mlp_body(x_ref, w1_ref, b1_ref, w2_ref, b2_ref, w3_ref, b3_ref,
              keep_ref, o_ref):
    x = x_ref[...]
    h = jax.lax.dot_general(x, w1_ref[...], (((1,), (1,)), ((), ())),
                            preferred_element_type=jnp.float32)
    h = jnp.maximum(h + b1_ref[...][None, :], 0.0)
    h = jax.lax.dot_general(h, w2_ref[...], (((1,), (1,)), ((), ())),
                            preferred_element_type=jnp.float32)
    h = jnp.maximum(h + b2_ref[...][None, :], 0.0)
    logits = jax.lax.dot_general(h, w3_ref[...], (((1,), (1,)), ((), ())),
                                 preferred_element_type=jnp.float32)
    logits = logits[:, 0:1] + b3_ref[0]
    o_ref[...] = jax.nn.sigmoid(logits) * keep_ref[...]


def _mlp(feats, w1, b1, w2, b2, w3, b3, keep):
    BM = 1024
    grid = (NPIX // BM,)
    return pl.pallas_call(
        _mlp_body,
        grid=grid,
        in_specs=[
            pl.BlockSpec((BM, DIM_FC), lambda i: (i, 0)),
            pl.BlockSpec((DIM_FC, DIM_FC), lambda i: (0, 0)),
            pl.BlockSpec((DIM_FC,), lambda i: (0,)),
            pl.BlockSpec((DIM_FC, DIM_FC), lambda i: (0, 0)),
            pl.BlockSpec((DIM_FC,), lambda i: (0,)),
            pl.BlockSpec((8, DIM_FC), lambda i: (0, 0)),
            pl.BlockSpec((1,), lambda i: (0,)),
            pl.BlockSpec((BM, 1), lambda i: (i, 0)),
        ],
        out_specs=pl.BlockSpec((BM, 1), lambda i: (i, 0)),
        out_shape=jax.ShapeDtypeStruct((NPIX, 1), jnp.float32),
    )(feats, w1, b1, w2, b2, w3, b3, keep)


def _maps_body(o_ref, lines_ref, nms_ref, joff_ref):
    o = o_ref[...]
    md0 = jax.nn.sigmoid(o[0])
    md1 = jax.nn.sigmoid(o[1])
    md2 = jax.nn.sigmoid(o[2])
    dis = jax.nn.sigmoid(o[3])
    m = jnp.maximum(o[5], o[6])
    e5 = jnp.exp(o[5] - m)
    e6 = jnp.exp(o[6] - m)
    jloc = e6 / (e5 + e6)
    joff_ref[0] = jax.nn.sigmoid(o[7]) - 0.5
    joff_ref[1] = jax.nn.sigmoid(o[8]) - 0.5

    y0 = jax.lax.broadcasted_iota(jnp.int32, (H, W), 0).astype(jnp.float32)
    x0 = jax.lax.broadcasted_iota(jnp.int32, (H, W), 1).astype(jnp.float32)
    md_ = (md0 - 0.5) * np.pi * 2.0
    st_ = md1 * np.pi / 2.0
    ed_ = -md2 * np.pi / 2.0
    cs_md = jnp.cos(md_)
    ss_md = jnp.sin(md_)
    cs_st = jnp.maximum(jnp.cos(st_), 0.001)
    ss_st = jnp.maximum(jnp.sin(st_), 0.001)
    cs_ed = jnp.maximum(jnp.cos(ed_), 0.001)
    ss_ed = jnp.minimum(jnp.sin(ed_), -0.001)
    y_st = ss_st / cs_st
    y_ed = ss_ed / cs_ed
    d = dis * 5.0
    lines_ref[0] = jnp.clip((cs_md - ss_md * y_st) * d + x0, 0.0, W - 1.0)
    lines_ref[1] = jnp.clip((ss_md + cs_md * y_st) * d + y0, 0.0, H - 1.0)
    lines_ref[2] = jnp.clip((cs_md - ss_md * y_ed) * d + x0, 0.0, W - 1.0)
    lines_ref[3] = jnp.clip((ss_md + cs_md * y_ed) * d + y0, 0.0, H - 1.0)

    ninf = jnp.float32(-jnp.inf)
    pad_row = jnp.full((1, W), ninf)
    up = jnp.concatenate([jloc[1:], pad_row], axis=0)
    dn = jnp.concatenate([pad_row, jloc[:-1]], axis=0)
    rm = jnp.maximum(jloc, jnp.maximum(up, dn))
    pad_col = jnp.full((H, 1), ninf)
    lf = jnp.concatenate([rm[:, 1:], pad_col], axis=1)
    rt = jnp.concatenate([pad_col, rm[:, :-1]], axis=1)
    ap = jnp.maximum(rm, jnp.maximum(lf, rt))
    nms_ref[...] = jloc * (jloc == ap).astype(jnp.float32)


def _maps(out0):
    return pl.pallas_call(
        _maps_body,
        out_shape=(
            jax.ShapeDtypeStruct((4, H, W), jnp.float32),
            jax.ShapeDtypeStruct((H, W), jnp.float32),
            jax.ShapeDtypeStruct((2, H, W), jnp.float32),
        ),
    )(out0)


def _topk_body(nms_ref, joff_ref, juncs_ref):
    fy = jax.lax.broadcasted_iota(jnp.int32, (H, W), 0)
    fx = jax.lax.broadcasted_iota(jnp.int32, (H, W), 1)
    flat = fy * W + fx
    joff0 = joff_ref[0]
    joff1 = joff_ref[1]
    big = jnp.int32(1 << 30)

    def step(k, v):
        mval = jnp.max(v)
        p = jnp.min(jnp.where(v == mval, flat, big))
        onehot = flat == p
        jox = jnp.sum(jnp.where(onehot, joff0, 0.0))
        joy = jnp.sum(jnp.where(onehot, joff1, 0.0))
        pxf = (p % W).astype(jnp.float32)
        pyf = (p // W).astype(jnp.float32)
        juncs_ref[0, k] = pxf + jox + 0.5
        juncs_ref[1, k] = pyf + joy + 0.5
        return jnp.where(onehot, -jnp.inf, v)

    lax.fori_loop(0, TOPK, step, nms_ref[...])


def _topk(nms, joff):
    return pl.pallas_call(
        _topk_body,
        out_shape=jax.ShapeDtypeStruct((2, 304), jnp.float32),
        out_specs=pl.BlockSpec(memory_space=pltpu.SMEM),
    )(nms, joff)


def _match_body(lines_ref, juncs_ref, uv_ref, keep_ref):
    l0 = lines_ref[0]
    l1 = lines_ref[1]
    l2 = lines_ref[2]
    l3 = lines_ref[3]
    inf = jnp.float32(jnp.inf)
    shp = (H, W)

    def step(j, c):
        d1m, i1, c1x, c1y, d2m, i2, c2x, c2y = c
        jx = juncs_ref[0, j]
        jy = juncs_ref[1, j]
        d1 = (l0 - jx) ** 2 + (l1 - jy) ** 2
        p1 = d1 < d1m
        d2 = (l2 - jx) ** 2 + (l3 - jy) ** 2
        p2 = d2 < d2m
        return (jnp.where(p1, d1, d1m), jnp.where(p1, j, i1),
                jnp.where(p1, jx, c1x), jnp.where(p1, jy, c1y),
                jnp.where(p2, d2, d2m), jnp.where(p2, j, i2),
                jnp.where(p2, jx, c2x), jnp.where(p2, jy, c2y))

    init = (jnp.full(shp, inf), jnp.zeros(shp, jnp.int32),
            jnp.zeros(shp), jnp.zeros(shp),
            jnp.full(shp, inf), jnp.zeros(shp, jnp.int32),
            jnp.zeros(shp), jnp.zeros(shp))
    _, i1, c1x, c1y, _, i2, c2x, c2y = lax.fori_loop(0, TOPK, step, init)
    swap = i2 < i1
    keep_ref[...] = (i1 != i2).astype(jnp.float32)
    uv_ref[0] = jnp.where(swap, c2x, c1x)
    uv_ref[1] = jnp.where(swap, c2y, c1y)
    uv_ref[2] = jnp.where(swap, c1x, c2x)
    uv_ref[3] = jnp.where(swap, c1y, c2y)


def _match(lines, juncs):
    return pl.pallas_call(
        _match_body,
        in_specs=[
            pl.BlockSpec(memory_space=pltpu.VMEM),
            pl.BlockSpec(memory_space=pltpu.SMEM),
        ],
        out_shape=(
            jax.ShapeDtypeStruct((4, H, W), jnp.float32),
            jax.ShapeDtypeStruct((H, W), jnp.float32),
        ),
    )(lines, juncs)


_NW = 32
_LPW = NPIX // _NW


def _sc_sample_body(uv_hbm, tsp_hbm, table_hbm, out_hbm,
                    uv_v, tsp_v, idx_v, w_v, g_v, o_v,
                    sem_g0, sem_g1, sem_o):
    nc = 2
    wid = lax.axis_index("s") * nc + lax.axis_index("c")
    base = wid * _LPW
    for c4 in range(4):
        pltpu.sync_copy(uv_hbm.at[pl.ds(c4 * NPIX + base, _LPW)],
                        uv_v.at[pl.ds(c4 * _LPW, _LPW)])
    pltpu.sync_copy(tsp_hbm, tsp_v)
    sems = (sem_g0, sem_g1)

    def calc_idx(l, b):
        ux = uv_v[pl.ds(0 * _LPW + l, 16)][0]
        uy = uv_v[pl.ds(1 * _LPW + l, 16)][0]
        vx = uv_v[pl.ds(2 * _LPW + l, 16)][0]
        vy = uv_v[pl.ds(3 * _LPW + l, 16)][0]
        for g in range(2):
            t = tsp_v[pl.ds(g * 16, 16)]
            omt = 1.0 - t
            px = ux * t + vx * omt - 0.5
            py = uy * t + vy * omt - 0.5
            px0i = jnp.clip(px.astype(jnp.int32), 0, W - 1)
            py0i = jnp.clip(py.astype(jnp.int32), 0, H - 1)
            px0 = px0i.astype(jnp.float32)
            py0 = py0i.astype(jnp.float32)
            px1 = jnp.minimum(px0 + 1.0, W - 1.0)
            py1 = jnp.minimum(py0 + 1.0, H - 1.0)
            px1i = px1.astype(jnp.int32)
            py1i = py1.astype(jnp.int32)
            wx0 = px1 - px
            wx1 = px - px0
            wy0 = py1 - py
            wy1 = py - py0
            r0 = py0i * W
            r1 = py1i * W
            idx_v[b, pl.ds(0 * 32 + g * 16, 16)] = r0 + px0i
            idx_v[b, pl.ds(1 * 32 + g * 16, 16)] = r1 + px0i
            idx_v[b, pl.ds(2 * 32 + g * 16, 16)] = r0 + px1i
            idx_v[b, pl.ds(3 * 32 + g * 16, 16)] = r1 + px1i
            w_v[b, 0, pl.ds(g * 16, 16)] = wy0 * wx0
            w_v[b, 1, pl.ds(g * 16, 16)] = wy1 * wx0
            w_v[b, 2, pl.ds(g * 16, 16)] = wy0 * wx1
            w_v[b, 3, pl.ds(g * 16, 16)] = wy1 * wx1

    def start_gather(b):
        pltpu.async_copy(table_hbm.at[idx_v.at[b]], g_v.at[b], sems[b])

    def wait_gather(b):
        pltpu.make_async_copy(table_hbm.at[idx_v.at[b]], g_v.at[b],
                              sems[b]).wait()

    def compute(b):
        for q in range(N_PTS1):
            acc = [None] * 8
            for k4 in range(4):
                pt = q * 4 + k4
                w00 = w_v[b, 0, pl.ds(pt, 16)][0]
                w10 = w_v[b, 1, pl.ds(pt, 16)][0]
                w01 = w_v[b, 2, pl.ds(pt, 16)][0]
                w11 = w_v[b, 3, pl.ds(pt, 16)][0]
                for v in range(8):
                    cs = pl.ds(v * 16, 16)
                    val = (g_v[b, 0 * 32 + pt, cs] * w00
                           + g_v[b, 1 * 32 + pt, cs] * w10
                           + g_v[b, 2 * 32 + pt, cs] * w01
                           + g_v[b, 3 * 32 + pt, cs] * w11)
                    acc[v] = val if acc[v] is None else jnp.maximum(acc[v], val)
            for v in range(8):
                o_v[pl.ds(b * DIM_FC + q * 128 + v * 16, 16)] = acc[v]

    def out_copy(i):
        return pltpu.make_async_copy(
            o_v, out_hbm.at[pl.ds((base + 2 * i) * DIM_FC, 2 * DIM_FC)],
            sem_o)

    npair = _LPW // 2
    calc_idx(0, 0)
    start_gather(0)

    def pair_body(i, carry):
        @pl.when(i > 0)
        def _():
            out_copy(i - 1).wait()
        calc_idx(2 * i + 1, 1)
        start_gather(1)
        wait_gather(0)
        compute(0)

        @pl.when(i < npair - 1)
        def _():
            calc_idx(2 * i + 2, 0)
            start_gather(0)
        wait_gather(1)
        compute(1)
        out_copy(i).start()
        return carry

    lax.fori_loop(0, npair, pair_body, 0)
    out_copy(npair - 1).wait()


def _sc_sample(uv, tspan, table):
    mesh = plsc.VectorSubcoreMesh(core_axis_name="c", subcore_axis_name="s")
    f = pl.kernel(
        _sc_sample_body,
        out_type=jax.ShapeDtypeStruct((NPIX * DIM_FC,), jnp.float32),
        mesh=mesh,
        scratch_types=[
            pltpu.VMEM((4 * _LPW + 32,), jnp.float32),
            pltpu.VMEM((N_PTS0,), jnp.float32),
            pltpu.VMEM((2, 4 * N_PTS0), jnp.int32),
            pltpu.VMEM((2, 4, 48), jnp.float32),
            pltpu.VMEM((2, 4 * N_PTS0, DIM_LOI), jnp.float32),
            pltpu.VMEM((2 * DIM_FC,), jnp.float32),
            pltpu.SemaphoreType.DMA,
            pltpu.SemaphoreType.DMA,
            pltpu.SemaphoreType.DMA,
        ],
    )
    return f(uv, tspan, table)


def kernel(output, features, fc1_w, fc1_b, w1, b1, w2, b2, w3, b3):
    features_flat = features[0].reshape(256, NPIX)
    loiT = _fc1(features_flat, fc1_w, fc1_b)

    lines, nms, joff = _maps(output[0])

    juncs = _topk(nms, joff)

    uv, iskeep = _match(lines, juncs)
    uv_flat = uv.reshape(-1)

    tspan = jnp.linspace(0.0, 1.0, N_PTS0)
    feats = _sc_sample(uv_flat, tspan, loiT).reshape(NPIX, DIM_FC)
    iskeep = iskeep.reshape(-1)

    w1p = w1.reshape(DIM_FC, DIM_LOI, N_PTS1).transpose(0, 2, 1).reshape(
        DIM_FC, DIM_FC)
    w3p = jnp.concatenate([w3, jnp.zeros((7, DIM_FC), jnp.float32)], axis=0)

    scores_out = _mlp(feats, w1p, b1, w2, b2, w3p, b3,
                      iskeep.reshape(NPIX, 1))
    return scores_out.reshape(-1)


# speedup vs baseline: 1.7681x; 1.3985x over previous
import functools

import jax
import jax.numpy as jnp
import numpy as np
from jax import lax
from jax.experimental import pallas as pl
from jax.experimental.pallas import tpu as pltpu
from jax.experimental.pallas import tpu_sc as plsc

H = W = 96
NPIX = H * W
DIM_LOI = 128
DIM_FC = 1024
N_PTS0 = 32
N_PTS1 = 8
TOPK = 300


def _fc1_body(f_ref, w_ref, b_ref, o_ref):
    a = f_ref[...]
    w = w_ref[...]
    o = jax.lax.dot_general(a, w, (((0,), (1,)), ((), ())),
                            preferred_element_type=jnp.float32)
    o_ref[...] = o + b_ref[...][None, :]


def _fc1(features_flat, fc1_w, fc1_b):
    BN = 1024
    grid = (NPIX // BN,)
    return pl.pallas_call(
        _fc1_body,
        grid=grid,
        in_specs=[
            pl.BlockSpec((256, BN), lambda i: (0, i)),
            pl.BlockSpec((DIM_LOI, 256), lambda i: (0, 0)),
            pl.BlockSpec((DIM_LOI,), lambda i: (0,)),
        ],
        out_specs=pl.BlockSpec((BN, DIM_LOI), lambda i: (i, 0)),
        out_shape=jax.ShapeDtypeStruct((NPIX, DIM_LOI), jnp.float32),
    )(features_flat, fc1_w, fc1_b)


def _mlp_body(x_ref, w1_ref, b1_ref, w2_ref, b2_ref, w3_ref, b3_ref,
              keep_ref, o_ref):
    x = x_ref[...]
    h = jax.lax.dot_general(x, w1_ref[...], (((1,), (1,)), ((), ())),
                            preferred_element_type=jnp.float32)
    h = jnp.maximum(h + b1_ref[...][None, :], 0.0)
    h = jax.lax.dot_general(h, w2_ref[...], (((1,), (1,)), ((), ())),
                            preferred_element_type=jnp.float32)
    h = jnp.maximum(h + b2_ref[...][None, :], 0.0)
    logits = jax.lax.dot_general(h, w3_ref[...], (((1,), (1,)), ((), ())),
                                 preferred_element_type=jnp.float32)
    logits = logits[:, 0:1] + b3_ref[0]
    o_ref[...] = jax.nn.sigmoid(logits) * keep_ref[...]


def _mlp(feats, w1, b1, w2, b2, w3, b3, keep):
    BM = 1024
    grid = (NPIX // BM,)
    return pl.pallas_call(
        _mlp_body,
        grid=grid,
        in_specs=[
            pl.BlockSpec((BM, DIM_FC), lambda i: (i, 0)),
            pl.BlockSpec((DIM_FC, DIM_FC), lambda i: (0, 0)),
            pl.BlockSpec((DIM_FC,), lambda i: (0,)),
            pl.BlockSpec((DIM_FC, DIM_FC), lambda i: (0, 0)),
            pl.BlockSpec((DIM_FC,), lambda i: (0,)),
            pl.BlockSpec((8, DIM_FC), lambda i: (0, 0)),
            pl.BlockSpec((1,), lambda i: (0,)),
            pl.BlockSpec((BM, 1), lambda i: (i, 0)),
        ],
        out_specs=pl.BlockSpec((BM, 1), lambda i: (i, 0)),
        out_shape=jax.ShapeDtypeStruct((NPIX, 1), jnp.float32),
    )(feats, w1, b1, w2, b2, w3, b3, keep)


def _maps_body(o_ref, lines_ref, nms_ref, joff_ref):
    o = o_ref[...]
    md0 = jax.nn.sigmoid(o[0])
    md1 = jax.nn.sigmoid(o[1])
    md2 = jax.nn.sigmoid(o[2])
    dis = jax.nn.sigmoid(o[3])
    m = jnp.maximum(o[5], o[6])
    e5 = jnp.exp(o[5] - m)
    e6 = jnp.exp(o[6] - m)
    jloc = e6 / (e5 + e6)
    joff_ref[0] = jax.nn.sigmoid(o[7]) - 0.5
    joff_ref[1] = jax.nn.sigmoid(o[8]) - 0.5

    y0 = jax.lax.broadcasted_iota(jnp.int32, (H, W), 0).astype(jnp.float32)
    x0 = jax.lax.broadcasted_iota(jnp.int32, (H, W), 1).astype(jnp.float32)
    md_ = (md0 - 0.5) * np.pi * 2.0
    st_ = md1 * np.pi / 2.0
    ed_ = -md2 * np.pi / 2.0
    cs_md = jnp.cos(md_)
    ss_md = jnp.sin(md_)
    cs_st = jnp.maximum(jnp.cos(st_), 0.001)
    ss_st = jnp.maximum(jnp.sin(st_), 0.001)
    cs_ed = jnp.maximum(jnp.cos(ed_), 0.001)
    ss_ed = jnp.minimum(jnp.sin(ed_), -0.001)
    y_st = ss_st / cs_st
    y_ed = ss_ed / cs_ed
    d = dis * 5.0
    lines_ref[0] = jnp.clip((cs_md - ss_md * y_st) * d + x0, 0.0, W - 1.0)
    lines_ref[1] = jnp.clip((ss_md + cs_md * y_st) * d + y0, 0.0, H - 1.0)
    lines_ref[2] = jnp.clip((cs_md - ss_md * y_ed) * d + x0, 0.0, W - 1.0)
    lines_ref[3] = jnp.clip((ss_md + cs_md * y_ed) * d + y0, 0.0, H - 1.0)

    ninf = jnp.float32(-jnp.inf)
    pad_row = jnp.full((1, W), ninf)
    up = jnp.concatenate([jloc[1:], pad_row], axis=0)
    dn = jnp.concatenate([pad_row, jloc[:-1]], axis=0)
    rm = jnp.maximum(jloc, jnp.maximum(up, dn))
    pad_col = jnp.full((H, 1), ninf)
    lf = jnp.concatenate([rm[:, 1:], pad_col], axis=1)
    rt = jnp.concatenate([pad_col, rm[:, :-1]], axis=1)
    ap = jnp.maximum(rm, jnp.maximum(lf, rt))
    nms_ref[...] = jloc * (jloc == ap).astype(jnp.float32)


def _maps(out0):
    return pl.pallas_call(
        _maps_body,
        out_shape=(
            jax.ShapeDtypeStruct((4, H, W), jnp.float32),
            jax.ShapeDtypeStruct((H, W), jnp.float32),
            jax.ShapeDtypeStruct((2, H, W), jnp.float32),
        ),
    )(out0)


def _topk_body(nms_ref, joff_ref, juncs_ref):
    fy = jax.lax.broadcasted_iota(jnp.int32, (H, W), 0)
    fx = jax.lax.broadcasted_iota(jnp.int32, (H, W), 1)
    flat = fy * W + fx
    joff0 = joff_ref[0]
    joff1 = joff_ref[1]
    big = jnp.int32(1 << 30)

    def step(k, v):
        mval = jnp.max(v)
        p = jnp.min(jnp.where(v == mval, flat, big))
        onehot = flat == p
        jox = jnp.sum(jnp.where(onehot, joff0, 0.0))
        joy = jnp.sum(jnp.where(onehot, joff1, 0.0))
        pxf = (p % W).astype(jnp.float32)
        pyf = (p // W).astype(jnp.float32)
        juncs_ref[0, k] = pxf + jox + 0.5
        juncs_ref[1, k] = pyf + joy + 0.5
        return jnp.where(onehot, -jnp.inf, v)

    lax.fori_loop(0, TOPK, step, nms_ref[...])


def _topk(nms, joff):
    return pl.pallas_call(
        _topk_body,
        out_shape=jax.ShapeDtypeStruct((2, 304), jnp.float32),
        out_specs=pl.BlockSpec(memory_space=pltpu.SMEM),
    )(nms, joff)


def _match_body(lines_ref, juncs_ref, uv_ref, keep_ref):
    l0 = lines_ref[0]
    l1 = lines_ref[1]
    l2 = lines_ref[2]
    l3 = lines_ref[3]
    inf = jnp.float32(jnp.inf)
    shp = (H, W)

    def step(j, c):
        d1m, i1, c1x, c1y, d2m, i2, c2x, c2y = c
        jx = juncs_ref[0, j]
        jy = juncs_ref[1, j]
        d1 = (l0 - jx) ** 2 + (l1 - jy) ** 2
        p1 = d1 < d1m
        d2 = (l2 - jx) ** 2 + (l3 - jy) ** 2
        p2 = d2 < d2m
        return (jnp.where(p1, d1, d1m), jnp.where(p1, j, i1),
                jnp.where(p1, jx, c1x), jnp.where(p1, jy, c1y),
                jnp.where(p2, d2, d2m), jnp.where(p2, j, i2),
                jnp.where(p2, jx, c2x), jnp.where(p2, jy, c2y))

    init = (jnp.full(shp, inf), jnp.zeros(shp, jnp.int32),
            jnp.zeros(shp), jnp.zeros(shp),
            jnp.full(shp, inf), jnp.zeros(shp, jnp.int32),
            jnp.zeros(shp), jnp.zeros(shp))
    _, i1, c1x, c1y, _, i2, c2x, c2y = lax.fori_loop(0, TOPK, step, init)
    swap = i2 < i1
    keep_ref[...] = (i1 != i2).astype(jnp.float32)
    uv_ref[0] = jnp.where(swap, c2x, c1x)
    uv_ref[1] = jnp.where(swap, c2y, c1y)
    uv_ref[2] = jnp.where(swap, c1x, c2x)
    uv_ref[3] = jnp.where(swap, c1y, c2y)


def _match(lines, juncs):
    return pl.pallas_call(
        _match_body,
        in_specs=[
            pl.BlockSpec(memory_space=pltpu.VMEM),
            pl.BlockSpec(memory_space=pltpu.SMEM),
        ],
        out_shape=(
            jax.ShapeDtypeStruct((4, H, W), jnp.float32),
            jax.ShapeDtypeStruct((H, W), jnp.float32),
        ),
    )(lines, juncs)


_NW = 32
_LPW = NPIX // _NW


def _sc_sample_body(uv_hbm, tsp_hbm, table_hbm, out_hbm,
                    uv_v, tsp_v, idx_v, w_v, g_v, o_v, ts_v,
                    sem_g0, sem_g1, sem_o):
    nc = 2
    sid = lax.axis_index("s")
    wid = sid * nc + lax.axis_index("c")
    base = wid * _LPW
    rpt = NPIX // 16
    pltpu.sync_copy(table_hbm.at[pl.ds(sid * rpt, rpt)],
                    ts_v.at[pl.ds(sid * rpt, rpt)])
    for c4 in range(4):
        pltpu.sync_copy(uv_hbm.at[pl.ds(c4 * NPIX + base, _LPW)],
                        uv_v.at[pl.ds(c4 * _LPW, _LPW)])
    pltpu.sync_copy(tsp_hbm, tsp_v)
    plsc.subcore_barrier()
    sems = (sem_g0, sem_g1)

    def calc_idx(l, b):
        ux = uv_v[pl.ds(0 * _LPW + l, 16)][0]
        uy = uv_v[pl.ds(1 * _LPW + l, 16)][0]
        vx = uv_v[pl.ds(2 * _LPW + l, 16)][0]
        vy = uv_v[pl.ds(3 * _LPW + l, 16)][0]
        for g in range(2):
            t = tsp_v[pl.ds(g * 16, 16)]
            omt = 1.0 - t
            px = ux * t + vx * omt - 0.5
            py = uy * t + vy * omt - 0.5
            px0i = jnp.clip(px.astype(jnp.int32), 0, W - 1)
            py0i = jnp.clip(py.astype(jnp.int32), 0, H - 1)
            px0 = px0i.astype(jnp.float32)
            py0 = py0i.astype(jnp.float32)
            px1 = jnp.minimum(px0 + 1.0, W - 1.0)
            py1 = jnp.minimum(py0 + 1.0, H - 1.0)
            px1i = px1.astype(jnp.int32)
            py1i = py1.astype(jnp.int32)
            wx0 = px1 - px
            wx1 = px - px0
            wy0 = py1 - py
            wy1 = py - py0
            r0 = py0i * W
            r1 = py1i * W
            idx_v[b, pl.ds(0 * 32 + g * 16, 16)] = r0 + px0i
            idx_v[b, pl.ds(1 * 32 + g * 16, 16)] = r1 + px0i
            idx_v[b, pl.ds(2 * 32 + g * 16, 16)] = r0 + px1i
            idx_v[b, pl.ds(3 * 32 + g * 16, 16)] = r1 + px1i
            w_v[b, 0, pl.ds(g * 16, 16)] = wy0 * wx0
            w_v[b, 1, pl.ds(g * 16, 16)] = wy1 * wx0
            w_v[b, 2, pl.ds(g * 16, 16)] = wy0 * wx1
            w_v[b, 3, pl.ds(g * 16, 16)] = wy1 * wx1

    def start_gather(b):
        pltpu.async_copy(ts_v.at[idx_v.at[b]], g_v.at[b], sems[b])

    def wait_gather(b):
        pltpu.make_async_copy(ts_v.at[idx_v.at[b]], g_v.at[b],
                              sems[b]).wait()

    def compute(b):
        for q in range(N_PTS1):
            acc = [None] * 8
            for k4 in range(4):
                pt = q * 4 + k4
                w00 = w_v[b, 0, pl.ds(pt, 16)][0]
                w10 = w_v[b, 1, pl.ds(pt, 16)][0]
                w01 = w_v[b, 2, pl.ds(pt, 16)][0]
                w11 = w_v[b, 3, pl.ds(pt, 16)][0]
                for v in range(8):
                    cs = pl.ds(v * 16, 16)
                    val = (g_v[b, 0 * 32 + pt, cs] * w00
                           + g_v[b, 1 * 32 + pt, cs] * w10
                           + g_v[b, 2 * 32 + pt, cs] * w01
                           + g_v[b, 3 * 32 + pt, cs] * w11)
                    acc[v] = val if acc[v] is None else jnp.maximum(acc[v], val)
            for v in range(8):
                o_v[pl.ds(b * DIM_FC + q * 128 + v * 16, 16)] = acc[v]

    def out_copy(i):
        return pltpu.make_async_copy(
            o_v, out_hbm.at[pl.ds((base + 2 * i) * DIM_FC, 2 * DIM_FC)],
            sem_o)

    npair = _LPW // 2
    calc_idx(0, 0)
    start_gather(0)

    def pair_body(i, carry):
        @pl.when(i > 0)
        def _():
            out_copy(i - 1).wait()
        calc_idx(2 * i + 1, 1)
        start_gather(1)
        wait_gather(0)
        compute(0)

        @pl.when(i < npair - 1)
        def _():
            calc_idx(2 * i + 2, 0)
            start_gather(0)
        wait_gather(1)
        compute(1)
        out_copy(i).start()
        return carry

    lax.fori_loop(0, npair, pair_body, 0)
    out_copy(npair - 1).wait()


def _sc_sample(uv, tspan, table):
    mesh = plsc.VectorSubcoreMesh(core_axis_name="c", subcore_axis_name="s")
    f = pl.kernel(
        _sc_sample_body,
        out_type=jax.ShapeDtypeStruct((NPIX * DIM_FC,), jnp.float32),
        mesh=mesh,
        scratch_types=[
            pltpu.VMEM((4 * _LPW + 32,), jnp.float32),
            pltpu.VMEM((N_PTS0,), jnp.float32),
            pltpu.VMEM((2, 4 * N_PTS0), jnp.int32),
            pltpu.VMEM((2, 4, 48), jnp.float32),
            pltpu.VMEM((2, 4 * N_PTS0, DIM_LOI), jnp.float32),
            pltpu.VMEM((2 * DIM_FC,), jnp.float32),
            pltpu.VMEM_SHARED((NPIX, DIM_LOI), jnp.float32),
            pltpu.SemaphoreType.DMA,
            pltpu.SemaphoreType.DMA,
            pltpu.SemaphoreType.DMA,
        ],
    )
    return f(uv, tspan, table)


def kernel(output, features, fc1_w, fc1_b, w1, b1, w2, b2, w3, b3):
    features_flat = features[0].reshape(256, NPIX)
    loiT = _fc1(features_flat, fc1_w, fc1_b)

    lines, nms, joff = _maps(output[0])

    juncs = _topk(nms, joff)

    uv, iskeep = _match(lines, juncs)
    uv_flat = uv.reshape(-1)

    tspan = jnp.linspace(0.0, 1.0, N_PTS0)
    feats = _sc_sample(uv_flat, tspan, loiT).reshape(NPIX, DIM_FC)
    iskeep = iskeep.reshape(-1)

    w1p = w1.reshape(DIM_FC, DIM_LOI, N_PTS1).transpose(0, 2, 1).reshape(
        DIM_FC, DIM_FC)
    w3p = jnp.concatenate([w3, jnp.zeros((7, DIM_FC), jnp.float32)], axis=0)

    scores_out = _mlp(feats, w1p, b1, w2, b2, w3p, b3,
                      iskeep.reshape(NPIX, 1))
    return scores_out.reshape(-1)


# speedup vs baseline: 1.8213x; 1.0301x over previous
import functools

import jax
import jax.numpy as jnp
import numpy as np
from jax import lax
from jax.experimental import pallas as pl
from jax.experimental.pallas import tpu as pltpu
from jax.experimental.pallas import tpu_sc as plsc

H = W = 96
NPIX = H * W
DIM_LOI = 128
DIM_FC = 1024
N_PTS0 = 32
N_PTS1 = 8
TOPK = 300


def _fc1_body(f_ref, w_ref, b_ref, o_ref):
    a = f_ref[...]
    w = w_ref[...]
    o = jax.lax.dot_general(a, w, (((0,), (1,)), ((), ())),
                            preferred_element_type=jnp.float32)
    o_ref[...] = o + b_ref[...][None, :]


def _fc1(features_flat, fc1_w, fc1_b):
    BN = 1024
    grid = (NPIX // BN,)
    return pl.pallas_call(
        _fc1_body,
        grid=grid,
        in_specs=[
            pl.BlockSpec((256, BN), lambda i: (0, i)),
            pl.BlockSpec((DIM_LOI, 256), lambda i: (0, 0)),
            pl.BlockSpec((DIM_LOI,), lambda i: (0,)),
        ],
        out_specs=pl.BlockSpec((BN, DIM_LOI), lambda i: (i, 0)),
        out_shape=jax.ShapeDtypeStruct((NPIX, DIM_LOI), jnp.float32),
    )(features_flat, fc1_w, fc1_b)


def _mlp_body(x_ref, w1_ref, b1_ref, w2_ref, b2_ref, w3_ref, b3_ref,
              keep_ref, o_ref):
    x = x_ref[...]
    h = jax.lax.dot_general(x, w1_ref[...], (((1,), (1,)), ((), ())),
                            preferred_element_type=jnp.float32)
    h = jnp.maximum(h + b1_ref[...][None, :], 0.0)
    h = jax.lax.dot_general(h, w2_ref[...], (((1,), (1,)), ((), ())),
                            preferred_element_type=jnp.float32)
    h = jnp.maximum(h + b2_ref[...][None, :], 0.0)
    logits = jax.lax.dot_general(h, w3_ref[...], (((1,), (1,)), ((), ())),
                                 preferred_element_type=jnp.float32)
    logits = logits[:, 0:1] + b3_ref[0]
    o_ref[...] = jax.nn.sigmoid(logits) * keep_ref[...]


def _mlp(feats, w1, b1, w2, b2, w3, b3, keep):
    n = feats.shape[0]
    BM = 512
    grid = (n // BM,)
    return pl.pallas_call(
        _mlp_body,
        grid=grid,
        in_specs=[
            pl.BlockSpec((BM, DIM_FC), lambda i: (i, 0)),
            pl.BlockSpec((DIM_FC, DIM_FC), lambda i: (0, 0)),
            pl.BlockSpec((DIM_FC,), lambda i: (0,)),
            pl.BlockSpec((DIM_FC, DIM_FC), lambda i: (0, 0)),
            pl.BlockSpec((DIM_FC,), lambda i: (0,)),
            pl.BlockSpec((8, DIM_FC), lambda i: (0, 0)),
            pl.BlockSpec((1,), lambda i: (0,)),
            pl.BlockSpec((BM, 1), lambda i: (i, 0)),
        ],
        out_specs=pl.BlockSpec((BM, 1), lambda i: (i, 0)),
        out_shape=jax.ShapeDtypeStruct((n, 1), jnp.float32),
    )(feats, w1, b1, w2, b2, w3, b3, keep)


def _maps_body(o_ref, lines_ref, nms_ref, joff_ref):
    o = o_ref[...]
    md0 = jax.nn.sigmoid(o[0])
    md1 = jax.nn.sigmoid(o[1])
    md2 = jax.nn.sigmoid(o[2])
    dis = jax.nn.sigmoid(o[3])
    m = jnp.maximum(o[5], o[6])
    e5 = jnp.exp(o[5] - m)
    e6 = jnp.exp(o[6] - m)
    jloc = e6 / (e5 + e6)
    joff_ref[0] = jax.nn.sigmoid(o[7]) - 0.5
    joff_ref[1] = jax.nn.sigmoid(o[8]) - 0.5

    y0 = jax.lax.broadcasted_iota(jnp.int32, (H, W), 0).astype(jnp.float32)
    x0 = jax.lax.broadcasted_iota(jnp.int32, (H, W), 1).astype(jnp.float32)
    md_ = (md0 - 0.5) * np.pi * 2.0
    st_ = md1 * np.pi / 2.0
    ed_ = -md2 * np.pi / 2.0
    cs_md = jnp.cos(md_)
    ss_md = jnp.sin(md_)
    cs_st = jnp.maximum(jnp.cos(st_), 0.001)
    ss_st = jnp.maximum(jnp.sin(st_), 0.001)
    cs_ed = jnp.maximum(jnp.cos(ed_), 0.001)
    ss_ed = jnp.minimum(jnp.sin(ed_), -0.001)
    y_st = ss_st / cs_st
    y_ed = ss_ed / cs_ed
    d = dis * 5.0
    lines_ref[0] = jnp.clip((cs_md - ss_md * y_st) * d + x0, 0.0, W - 1.0)
    lines_ref[1] = jnp.clip((ss_md + cs_md * y_st) * d + y0, 0.0, H - 1.0)
    lines_ref[2] = jnp.clip((cs_md - ss_md * y_ed) * d + x0, 0.0, W - 1.0)
    lines_ref[3] = jnp.clip((ss_md + cs_md * y_ed) * d + y0, 0.0, H - 1.0)

    ninf = jnp.float32(-jnp.inf)
    pad_row = jnp.full((1, W), ninf)
    up = jnp.concatenate([jloc[1:], pad_row], axis=0)
    dn = jnp.concatenate([pad_row, jloc[:-1]], axis=0)
    rm = jnp.maximum(jloc, jnp.maximum(up, dn))
    pad_col = jnp.full((H, 1), ninf)
    lf = jnp.concatenate([rm[:, 1:], pad_col], axis=1)
    rt = jnp.concatenate([pad_col, rm[:, :-1]], axis=1)
    ap = jnp.maximum(rm, jnp.maximum(lf, rt))
    nms_ref[...] = jloc * (jloc == ap).astype(jnp.float32)


def _maps(out0):
    return pl.pallas_call(
        _maps_body,
        out_shape=(
            jax.ShapeDtypeStruct((4, H, W), jnp.float32),
            jax.ShapeDtypeStruct((H, W), jnp.float32),
            jax.ShapeDtypeStruct((2, H, W), jnp.float32),
        ),
    )(out0)


def _topk_body(nms_ref, joff_ref, juncs_ref):
    fy = jax.lax.broadcasted_iota(jnp.int32, (H, W), 0)
    fx = jax.lax.broadcasted_iota(jnp.int32, (H, W), 1)
    flat = fy * W + fx
    joff0 = joff_ref[0]
    joff1 = joff_ref[1]
    big = jnp.int32(1 << 30)

    def step(k, v):
        mval = jnp.max(v)
        p = jnp.min(jnp.where(v == mval, flat, big))
        onehot = flat == p
        jox = jnp.sum(jnp.where(onehot, joff0, 0.0))
        joy = jnp.sum(jnp.where(onehot, joff1, 0.0))
        pxf = (p % W).astype(jnp.float32)
        pyf = (p // W).astype(jnp.float32)
        juncs_ref[0, k] = pxf + jox + 0.5
        juncs_ref[1, k] = pyf + joy + 0.5
        return jnp.where(onehot, -jnp.inf, v)

    lax.fori_loop(0, TOPK, step, nms_ref[...])


def _topk(nms, joff):
    return pl.pallas_call(
        _topk_body,
        out_shape=jax.ShapeDtypeStruct((2, 304), jnp.float32),
        out_specs=pl.BlockSpec(memory_space=pltpu.SMEM),
    )(nms, joff)


def _match_body(lines_ref, juncs_ref, uv_ref, keep_ref):
    l0 = lines_ref[0]
    l1 = lines_ref[1]
    l2 = lines_ref[2]
    l3 = lines_ref[3]
    inf = jnp.float32(jnp.inf)
    shp = (H, W)

    def step(j, c):
        d1m, i1, c1x, c1y, d2m, i2, c2x, c2y = c
        jx = juncs_ref[0, j]
        jy = juncs_ref[1, j]
        d1 = (l0 - jx) ** 2 + (l1 - jy) ** 2
        p1 = d1 < d1m
        d2 = (l2 - jx) ** 2 + (l3 - jy) ** 2
        p2 = d2 < d2m
        return (jnp.where(p1, d1, d1m), jnp.where(p1, j, i1),
                jnp.where(p1, jx, c1x), jnp.where(p1, jy, c1y),
                jnp.where(p2, d2, d2m), jnp.where(p2, j, i2),
                jnp.where(p2, jx, c2x), jnp.where(p2, jy, c2y))

    init = (jnp.full(shp, inf), jnp.zeros(shp, jnp.int32),
            jnp.zeros(shp), jnp.zeros(shp),
            jnp.full(shp, inf), jnp.zeros(shp, jnp.int32),
            jnp.zeros(shp), jnp.zeros(shp))
    _, i1, c1x, c1y, _, i2, c2x, c2y = lax.fori_loop(0, TOPK, step, init)
    swap = i2 < i1
    keep_ref[...] = (i1 != i2).astype(jnp.float32)
    uv_ref[0] = jnp.where(swap, c2x, c1x)
    uv_ref[1] = jnp.where(swap, c2y, c1y)
    uv_ref[2] = jnp.where(swap, c1x, c2x)
    uv_ref[3] = jnp.where(swap, c1y, c2y)


def _match(lines, juncs):
    return pl.pallas_call(
        _match_body,
        in_specs=[
            pl.BlockSpec(memory_space=pltpu.VMEM),
            pl.BlockSpec(memory_space=pltpu.SMEM),
        ],
        out_shape=(
            jax.ShapeDtypeStruct((4, H, W), jnp.float32),
            jax.ShapeDtypeStruct((H, W), jnp.float32),
        ),
    )(lines, juncs)


_NW = 32
_NHALF = NPIX // 2
_LPW = _NHALF // _NW


def _sc_sample_body(half, uv_hbm, tsp_hbm, table_hbm, out_hbm,
                    uv_v, tsp_v, idx_v, w_v, g_v, o_v, ts_v,
                    sem_g0, sem_g1, sem_o):
    nc = 2
    sid = lax.axis_index("s")
    wid = sid * nc + lax.axis_index("c")
    base = wid * _LPW
    base_uv = half * _NHALF + base
    rpt = NPIX // 16
    pltpu.sync_copy(table_hbm.at[pl.ds(sid * rpt, rpt)],
                    ts_v.at[pl.ds(sid * rpt, rpt)])
    for c4 in range(4):
        pltpu.sync_copy(uv_hbm.at[pl.ds(c4 * NPIX + base_uv, _LPW)],
                        uv_v.at[pl.ds(c4 * _LPW, _LPW)])
    pltpu.sync_copy(tsp_hbm, tsp_v)
    plsc.subcore_barrier()
    sems = (sem_g0, sem_g1)

    def calc_idx(l, b):
        ux = uv_v[pl.ds(0 * _LPW + l, 16)][0]
        uy = uv_v[pl.ds(1 * _LPW + l, 16)][0]
        vx = uv_v[pl.ds(2 * _LPW + l, 16)][0]
        vy = uv_v[pl.ds(3 * _LPW + l, 16)][0]
        for g in range(2):
            t = tsp_v[pl.ds(g * 16, 16)]
            omt = 1.0 - t
            px = ux * t + vx * omt - 0.5
            py = uy * t + vy * omt - 0.5
            px0i = jnp.clip(px.astype(jnp.int32), 0, W - 1)
            py0i = jnp.clip(py.astype(jnp.int32), 0, H - 1)
            px0 = px0i.astype(jnp.float32)
            py0 = py0i.astype(jnp.float32)
            px1 = jnp.minimum(px0 + 1.0, W - 1.0)
            py1 = jnp.minimum(py0 + 1.0, H - 1.0)
            px1i = px1.astype(jnp.int32)
            py1i = py1.astype(jnp.int32)
            wx0 = px1 - px
            wx1 = px - px0
            wy0 = py1 - py
            wy1 = py - py0
            r0 = py0i * W
            r1 = py1i * W
            idx_v[b, pl.ds(0 * 32 + g * 16, 16)] = r0 + px0i
            idx_v[b, pl.ds(1 * 32 + g * 16, 16)] = r1 + px0i
            idx_v[b, pl.ds(2 * 32 + g * 16, 16)] = r0 + px1i
            idx_v[b, pl.ds(3 * 32 + g * 16, 16)] = r1 + px1i
            w_v[b, 0, pl.ds(g * 16, 16)] = wy0 * wx0
            w_v[b, 1, pl.ds(g * 16, 16)] = wy1 * wx0
            w_v[b, 2, pl.ds(g * 16, 16)] = wy0 * wx1
            w_v[b, 3, pl.ds(g * 16, 16)] = wy1 * wx1

    def start_gather(b):
        pltpu.async_copy(ts_v.at[idx_v.at[b]], g_v.at[b], sems[b])

    def wait_gather(b):
        pltpu.make_async_copy(ts_v.at[idx_v.at[b]], g_v.at[b],
                              sems[b]).wait()

    def compute(b):
        for q in range(N_PTS1):
            acc = [None] * 8
            for k4 in range(4):
                pt = q * 4 + k4
                w00 = w_v[b, 0, pl.ds(pt, 16)][0]
                w10 = w_v[b, 1, pl.ds(pt, 16)][0]
                w01 = w_v[b, 2, pl.ds(pt, 16)][0]
                w11 = w_v[b, 3, pl.ds(pt, 16)][0]
                for v in range(8):
                    cs = pl.ds(v * 16, 16)
                    val = (g_v[b, 0 * 32 + pt, cs] * w00
                           + g_v[b, 1 * 32 + pt, cs] * w10
                           + g_v[b, 2 * 32 + pt, cs] * w01
                           + g_v[b, 3 * 32 + pt, cs] * w11)
                    acc[v] = val if acc[v] is None else jnp.maximum(acc[v], val)
            for v in range(8):
                o_v[pl.ds(b * DIM_FC + q * 128 + v * 16, 16)] = acc[v]

    def out_copy(i):
        return pltpu.make_async_copy(
            o_v, out_hbm.at[pl.ds((base + 2 * i) * DIM_FC, 2 * DIM_FC)],
            sem_o)

    npair = _LPW // 2
    calc_idx(0, 0)
    start_gather(0)

    def pair_body(i, carry):
        @pl.when(i > 0)
        def _():
            out_copy(i - 1).wait()
        calc_idx(2 * i + 1, 1)
        start_gather(1)
        wait_gather(0)
        compute(0)

        @pl.when(i < npair - 1)
        def _():
            calc_idx(2 * i + 2, 0)
            start_gather(0)
        wait_gather(1)
        compute(1)
        out_copy(i).start()
        return carry

    lax.fori_loop(0, npair, pair_body, 0)
    out_copy(npair - 1).wait()


def _sc_sample(uv, tspan, table, half):
    mesh = plsc.VectorSubcoreMesh(core_axis_name="c", subcore_axis_name="s")
    f = pl.kernel(
        functools.partial(_sc_sample_body, half),
        out_type=jax.ShapeDtypeStruct((_NHALF * DIM_FC,), jnp.float32),
        mesh=mesh,
        scratch_types=[
            pltpu.VMEM((4 * _LPW + 32,), jnp.float32),
            pltpu.VMEM((N_PTS0,), jnp.float32),
            pltpu.VMEM((2, 4 * N_PTS0), jnp.int32),
            pltpu.VMEM((2, 4, 48), jnp.float32),
            pltpu.VMEM((2, 4 * N_PTS0, DIM_LOI), jnp.float32),
            pltpu.VMEM((2 * DIM_FC,), jnp.float32),
            pltpu.VMEM_SHARED((NPIX, DIM_LOI), jnp.float32),
            pltpu.SemaphoreType.DMA,
            pltpu.SemaphoreType.DMA,
            pltpu.SemaphoreType.DMA,
        ],
    )
    return f(uv, tspan, table)


def kernel(output, features, fc1_w, fc1_b, w1, b1, w2, b2, w3, b3):
    features_flat = features[0].reshape(256, NPIX)
    loiT = _fc1(features_flat, fc1_w, fc1_b)

    lines, nms, joff = _maps(output[0])

    juncs = _topk(nms, joff)

    uv, iskeep = _match(lines, juncs)
    uv_flat = uv.reshape(-1)

    tspan = jnp.linspace(0.0, 1.0, N_PTS0)
    feats0 = _sc_sample(uv_flat, tspan, loiT, 0).reshape(_NHALF, DIM_FC)
    feats1 = _sc_sample(uv_flat, tspan, loiT, 1).reshape(_NHALF, DIM_FC)
    iskeep = iskeep.reshape(-1)

    w1p = w1.reshape(DIM_FC, DIM_LOI, N_PTS1).transpose(0, 2, 1).reshape(
        DIM_FC, DIM_FC)
    w3p = jnp.concatenate([w3, jnp.zeros((7, DIM_FC), jnp.float32)], axis=0)

    keep2 = iskeep.reshape(NPIX, 1)
    s0 = _mlp(feats0, w1p, b1, w2, b2, w3p, b3, keep2[:_NHALF])
    s1 = _mlp(feats1, w1p, b1, w2, b2, w3p, b3, keep2[_NHALF:])
    return jnp.concatenate([s0, s1], axis=0).reshape(-1)
